# baseline (speedup 1.0000x reference)
import jax
import jax.numpy as jnp
from jax.experimental import pallas as pl
from jax.experimental.pallas import tpu as pltpu

N_TOK = 32768
FEAT_DIM = 768
TEXT_DIM = 768
NUM_EXPERTS = 8
HID = 128
BLK = 2048


def _reparam_body(fmu_ref, frho_ref, tmu_ref, trho_ref, cmu_ref, crho_ref,
                  ef_ref, et_ref, ec_ref, fw_ref, tw_ref, cw_ref):
    fw_ref[...] = fmu_ref[...] + jnp.log(1.0 + jnp.exp(frho_ref[...])) * ef_ref[...]
    tw_ref[...] = tmu_ref[...] + jnp.log(1.0 + jnp.exp(trho_ref[...])) * et_ref[...]
    cw_ref[...] = cmu_ref[...] + jnp.log(1.0 + jnp.exp(crho_ref[...])) * ec_ref[...]


def _router_body(temp_ref, f_lo_ref, f_hi_ref, t_lo_ref, t_hi_ref,
                 fw_ref, tw_ref, cw_ref, probs_ref, logits_ref):
    half = FEAT_DIM // 2
    fp = (jnp.dot(f_lo_ref[...], fw_ref[:half, :], preferred_element_type=jnp.float32)
          + jnp.dot(f_hi_ref[...], fw_ref[half:, :], preferred_element_type=jnp.float32))
    tp = (jnp.dot(t_lo_ref[...], tw_ref[:half, :], preferred_element_type=jnp.float32)
          + jnp.dot(t_hi_ref[...], tw_ref[half:, :], preferred_element_type=jnp.float32))
    logits = (jnp.dot(fp, cw_ref[:HID, :], preferred_element_type=jnp.float32)
              + jnp.dot(tp, cw_ref[HID:, :], preferred_element_type=jnp.float32))
    inv_t = 1.0 / jnp.maximum(temp_ref[0, 0], 0.1)
    logits = logits * inv_t
    logits_ref[...] = logits
    m = jnp.max(logits, axis=1, keepdims=True)
    e = jnp.exp(logits - m)
    probs_ref[...] = e / jnp.sum(e, axis=1, keepdims=True)


def kernel(feature, text_embedding, feature_mu, feature_rho, text_mu, text_rho,
           combined_mu, combined_rho, temperature, epsilon_f, epsilon_t, epsilon_c):
    fw, tw, cw = pl.pallas_call(
        _reparam_body,
        out_shape=[
            jax.ShapeDtypeStruct((FEAT_DIM, HID), jnp.float32),
            jax.ShapeDtypeStruct((TEXT_DIM, HID), jnp.float32),
            jax.ShapeDtypeStruct((2 * HID, NUM_EXPERTS), jnp.float32),
        ],
    )(feature_mu, feature_rho, text_mu, text_rho, combined_mu, combined_rho,
      epsilon_f, epsilon_t, epsilon_c)

    temp2d = temperature.reshape(1, 1)
    full = lambda shape: pl.BlockSpec(shape, lambda i: (0, 0))
    grid = N_TOK // BLK
    probs, logits = pl.pallas_call(
        _router_body,
        grid=(grid,),
        in_specs=[
            full((1, 1)),
            pl.BlockSpec((BLK, FEAT_DIM // 2), lambda i: (i, 0)),
            pl.BlockSpec((BLK, FEAT_DIM // 2), lambda i: (i, 1)),
            pl.BlockSpec((BLK, TEXT_DIM // 2), lambda i: (i, 0)),
            pl.BlockSpec((BLK, TEXT_DIM // 2), lambda i: (i, 1)),
            full((FEAT_DIM, HID)),
            full((TEXT_DIM, HID)),
            full((2 * HID, NUM_EXPERTS)),
        ],
        out_specs=[
            pl.BlockSpec((BLK, NUM_EXPERTS), lambda i: (i, 0)),
            pl.BlockSpec((BLK, NUM_EXPERTS), lambda i: (i, 0)),
        ],
        out_shape=[
            jax.ShapeDtypeStruct((N_TOK, NUM_EXPERTS), jnp.float32),
            jax.ShapeDtypeStruct((N_TOK, NUM_EXPERTS), jnp.float32),
        ],
        compiler_params=pltpu.CompilerParams(
            dimension_semantics=("arbitrary",),
            vmem_limit_bytes=120 * 1024 * 1024,
        ),
    )(temp2d, feature, feature, text_embedding, text_embedding, fw, tw, cw)
    return (probs, logits)


# speedup vs baseline: 1.0494x; 1.0494x over previous
import jax
import jax.numpy as jnp
from jax.experimental import pallas as pl
from jax.experimental.pallas import tpu as pltpu

N_TOK = 32768
FEAT_DIM = 768
TEXT_DIM = 768
NUM_EXPERTS = 8
HID = 128
SUB = 1024
DEPTH = 4
NSTEPS = N_TOK // SUB


def _router_body(temp_ref, f_hbm, t_hbm, fmu_ref, frho_ref, tmu_ref, trho_ref,
                 cmu_ref, crho_ref, ef_ref, et_ref, ec_ref,
                 probs_ref, logits_ref,
                 fbuf, tbuf, fw_s, tw_s, cw_s, sems):
    i = pl.program_id(0)

    def _fcopy(blk, slot):
        return pltpu.make_async_copy(
            f_hbm.at[pl.ds(blk * SUB, SUB), :], fbuf.at[slot], sems.at[0, slot])

    def _tcopy(blk, slot):
        return pltpu.make_async_copy(
            t_hbm.at[pl.ds(blk * SUB, SUB), :], tbuf.at[slot], sems.at[1, slot])

    @pl.when(i == 0)
    def _():
        for d in range(DEPTH):
            _fcopy(d, d).start()
            _tcopy(d, d).start()
        fw_s[...] = fmu_ref[...] + jnp.log(1.0 + jnp.exp(frho_ref[...])) * ef_ref[...]
        tw_s[...] = tmu_ref[...] + jnp.log(1.0 + jnp.exp(trho_ref[...])) * et_ref[...]
        cw_s[...] = cmu_ref[...] + jnp.log(1.0 + jnp.exp(crho_ref[...])) * ec_ref[...]

    slot = jax.lax.rem(i, DEPTH)
    _fcopy(i, slot).wait()
    _tcopy(i, slot).wait()

    fp = jnp.dot(fbuf[slot], fw_s[...], preferred_element_type=jnp.float32)
    tp = jnp.dot(tbuf[slot], tw_s[...], preferred_element_type=jnp.float32)
    logits = (jnp.dot(fp, cw_s[:HID, :], preferred_element_type=jnp.float32)
              + jnp.dot(tp, cw_s[HID:, :], preferred_element_type=jnp.float32))
    inv_t = 1.0 / jnp.maximum(temp_ref[0, 0], 0.1)
    logits = logits * inv_t
    logits_ref[...] = logits
    m = jnp.max(logits, axis=1, keepdims=True)
    e = jnp.exp(logits - m)
    probs_ref[...] = e / jnp.sum(e, axis=1, keepdims=True)

    nxt = i + DEPTH

    @pl.when(nxt < NSTEPS)
    def _():
        _fcopy(nxt, slot).start()
        _tcopy(nxt, slot).start()


def kernel(feature, text_embedding, feature_mu, feature_rho, text_mu, text_rho,
           combined_mu, combined_rho, temperature, epsilon_f, epsilon_t, epsilon_c):
    temp2d = temperature.reshape(1, 1)
    full = lambda shape: pl.BlockSpec(shape, lambda i: (0, 0))
    hbm = pl.BlockSpec(memory_space=pltpu.MemorySpace.HBM)
    probs, logits = pl.pallas_call(
        _router_body,
        grid=(NSTEPS,),
        in_specs=[
            full((1, 1)),
            hbm,
            hbm,
            full((FEAT_DIM, HID)),
            full((FEAT_DIM, HID)),
            full((TEXT_DIM, HID)),
            full((TEXT_DIM, HID)),
            full((2 * HID, NUM_EXPERTS)),
            full((2 * HID, NUM_EXPERTS)),
            full((FEAT_DIM, HID)),
            full((TEXT_DIM, HID)),
            full((2 * HID, NUM_EXPERTS)),
        ],
        out_specs=[
            pl.BlockSpec((SUB, NUM_EXPERTS), lambda i: (i, 0)),
            pl.BlockSpec((SUB, NUM_EXPERTS), lambda i: (i, 0)),
        ],
        out_shape=[
            jax.ShapeDtypeStruct((N_TOK, NUM_EXPERTS), jnp.float32),
            jax.ShapeDtypeStruct((N_TOK, NUM_EXPERTS), jnp.float32),
        ],
        scratch_shapes=[
            pltpu.VMEM((DEPTH, SUB, FEAT_DIM), jnp.float32),
            pltpu.VMEM((DEPTH, SUB, TEXT_DIM), jnp.float32),
            pltpu.VMEM((FEAT_DIM, HID), jnp.float32),
            pltpu.VMEM((TEXT_DIM, HID), jnp.float32),
            pltpu.VMEM((2 * HID, NUM_EXPERTS), jnp.float32),
            pltpu.SemaphoreType.DMA((2, DEPTH)),
        ],
        compiler_params=pltpu.CompilerParams(
            dimension_semantics=("arbitrary",),
            vmem_limit_bytes=120 * 1024 * 1024,
        ),
    )(temp2d, feature, text_embedding, feature_mu, feature_rho, text_mu,
      text_rho, combined_mu, combined_rho, epsilon_f, epsilon_t, epsilon_c)
    return (probs, logits)
